# initial kernel scaffold (unmeasured)
import jax
import jax.numpy as jnp
from jax import lax
from jax.experimental import pallas as pl
from jax.experimental.pallas import tpu as pltpu


def _partial_matmul(A, B):
    M, K = A.shape
    _, N = B.shape
    bm, bn = 512, 1024

    def body(a_ref, b_ref, o_ref):
        o_ref[...] = jnp.dot(
            a_ref[...], b_ref[...], preferred_element_type=jnp.float32
        )

    return pl.pallas_call(
        body,
        grid=(M // bm, N // bn),
        in_specs=[
            pl.BlockSpec((bm, K), lambda i, j: (i, 0)),
            pl.BlockSpec((K, bn), lambda i, j: (0, j)),
        ],
        out_specs=pl.BlockSpec((bm, bn), lambda i, j: (i, j)),
        out_shape=jax.ShapeDtypeStruct((M, N), jnp.float32),
    )(A, B)


def _allreduce_x(partial):
    M, N = partial.shape
    bm = 512
    n_tiles = M // bm

    def body(p_ref, o_ref, recv_hbm, mine_v, theirs_v, out_v,
             cp_sems, send_sem, recv_sem):
        my_x = lax.axis_index("x")
        my_y = lax.axis_index("y")
        peer = (1 - my_x, my_y)

        barrier = pltpu.get_barrier_semaphore()
        pl.semaphore_signal(
            barrier, inc=1, device_id=peer,
            device_id_type=pl.DeviceIdType.MESH,
        )
        pl.semaphore_wait(barrier, 1)

        rdma = pltpu.make_async_remote_copy(
            src_ref=p_ref,
            dst_ref=recv_hbm,
            send_sem=send_sem,
            recv_sem=recv_sem,
            device_id=peer,
            device_id_type=pl.DeviceIdType.MESH,
        )
        rdma.start()
        rdma.wait()

        for i in range(n_tiles):
            rows = pl.ds(i * bm, bm)
            ld_a = pltpu.make_async_copy(p_ref.at[rows], mine_v, cp_sems.at[0])
            ld_b = pltpu.make_async_copy(
                recv_hbm.at[rows], theirs_v, cp_sems.at[1]
            )
            ld_a.start()
            ld_b.start()
            ld_a.wait()
            ld_b.wait()
            out_v[...] = mine_v[...] + theirs_v[...]
            st = pltpu.make_async_copy(out_v, o_ref.at[rows], cp_sems.at[2])
            st.start()
            st.wait()

    return pl.pallas_call(
        body,
        in_specs=[pl.BlockSpec(memory_space=pltpu.MemorySpace.HBM)],
        out_specs=pl.BlockSpec(memory_space=pltpu.MemorySpace.HBM),
        out_shape=jax.ShapeDtypeStruct((M, N), jnp.float32),
        scratch_shapes=[
            pltpu.MemorySpace.HBM((M, N), jnp.float32),
            pltpu.MemorySpace.VMEM((bm, N), jnp.float32),
            pltpu.MemorySpace.VMEM((bm, N), jnp.float32),
            pltpu.MemorySpace.VMEM((bm, N), jnp.float32),
            pltpu.SemaphoreType.DMA((3,)),
            pltpu.SemaphoreType.DMA,
            pltpu.SemaphoreType.DMA,
        ],
        compiler_params=pltpu.CompilerParams(collective_id=0),
    )(partial)


def kernel(A, B):
    partial = _partial_matmul(A, B)
    return _allreduce_x(partial)


# baseline (device time: 975051 ns/iter reference)
import jax
import jax.numpy as jnp
from jax import lax
from jax.experimental import pallas as pl
from jax.experimental.pallas import tpu as pltpu


def _partial_matmul(A, B):
    M, K = A.shape
    _, N = B.shape
    bm, bn = 512, 1024

    def body(a_ref, b_ref, o_ref):
        o_ref[...] = jnp.dot(
            a_ref[...], b_ref[...], preferred_element_type=jnp.float32
        )

    return pl.pallas_call(
        body,
        grid=(M // bm, N // bn),
        in_specs=[
            pl.BlockSpec((bm, K), lambda i, j: (i, 0)),
            pl.BlockSpec((K, bn), lambda i, j: (0, j)),
        ],
        out_specs=pl.BlockSpec((bm, bn), lambda i, j: (i, j)),
        out_shape=jax.ShapeDtypeStruct((M, N), jnp.float32),
    )(A, B)


def _allreduce_x(partial):
    M, N = partial.shape
    bm = 512
    n_tiles = M // bm

    def body(p_ref, o_ref, recv_hbm, mine_v, theirs_v, out_v,
             cp_sems, send_sem, recv_sem):
        my_x = lax.axis_index("x")
        my_y = lax.axis_index("y")
        peer = (1 - my_x, my_y)

        barrier = pltpu.get_barrier_semaphore()
        pl.semaphore_signal(
            barrier, inc=1, device_id=peer,
            device_id_type=pl.DeviceIdType.MESH,
        )
        pl.semaphore_wait(barrier, 1)

        rdma = pltpu.make_async_remote_copy(
            src_ref=p_ref,
            dst_ref=recv_hbm,
            send_sem=send_sem,
            recv_sem=recv_sem,
            device_id=peer,
            device_id_type=pl.DeviceIdType.MESH,
        )
        rdma.start()
        rdma.wait()

        for i in range(n_tiles):
            rows = pl.ds(i * bm, bm)
            ld_a = pltpu.make_async_copy(p_ref.at[rows], mine_v, cp_sems.at[0])
            ld_b = pltpu.make_async_copy(
                recv_hbm.at[rows], theirs_v, cp_sems.at[1]
            )
            ld_a.start()
            ld_b.start()
            ld_a.wait()
            ld_b.wait()
            out_v[...] = mine_v[...] + theirs_v[...]
            st = pltpu.make_async_copy(out_v, o_ref.at[rows], cp_sems.at[2])
            st.start()
            st.wait()

    out, _recv = pl.pallas_call(
        body,
        in_specs=[pl.BlockSpec(memory_space=pltpu.MemorySpace.HBM)],
        out_specs=[
            pl.BlockSpec(memory_space=pltpu.MemorySpace.HBM),
            pl.BlockSpec(memory_space=pltpu.MemorySpace.HBM),
        ],
        out_shape=[
            jax.ShapeDtypeStruct((M, N), jnp.float32),
            jax.ShapeDtypeStruct((M, N), jnp.float32),
        ],
        scratch_shapes=[
            pltpu.MemorySpace.VMEM((bm, N), jnp.float32),
            pltpu.MemorySpace.VMEM((bm, N), jnp.float32),
            pltpu.MemorySpace.VMEM((bm, N), jnp.float32),
            pltpu.SemaphoreType.DMA((3,)),
            pltpu.SemaphoreType.DMA,
            pltpu.SemaphoreType.DMA,
        ],
        compiler_params=pltpu.CompilerParams(collective_id=0),
    )(partial)
    return out


def kernel(A, B):
    partial = _partial_matmul(A, B)
    return _allreduce_x(partial)


# device time: 449170 ns/iter; 2.1708x vs baseline; 2.1708x over previous
import jax
import jax.numpy as jnp
from jax import lax
from jax.experimental import pallas as pl
from jax.experimental.pallas import tpu as pltpu

BM = 128


def kernel(A, B):
    M, K = A.shape
    _, N = B.shape
    half = M // 2
    nt = half // BM

    def body(a_ref, b_ref, o_ref, recv_x, a_v, pt_v, th_v, ys_v,
             a_sems, th_sem, st_sems, sx_sems, rx_sems, sy_sems, ry_sems):
        my_x = lax.axis_index("x")
        my_y = lax.axis_index("y")
        peer_x = (1 - my_x, my_y)
        peer_y = (my_x, 1 - my_y)
        half_start = my_y * half

        barrier = pltpu.get_barrier_semaphore()
        for peer in (peer_x, peer_y):
            pl.semaphore_signal(
                barrier, inc=1, device_id=peer,
                device_id_type=pl.DeviceIdType.MESH,
            )
        pl.semaphore_wait(barrier, 2)

        def a_rows(i):
            return pl.ds(half_start + i * BM, BM)

        def x_rdma(i):
            return pltpu.make_async_remote_copy(
                src_ref=pt_v.at[i % 2],
                dst_ref=recv_x.at[pl.ds(i * BM, BM)],
                send_sem=sx_sems.at[i % 2],
                recv_sem=rx_sems.at[i],
                device_id=peer_x,
                device_id_type=pl.DeviceIdType.MESH,
            )

        def y_rdma(i):
            return pltpu.make_async_remote_copy(
                src_ref=ys_v.at[i % 2],
                dst_ref=o_ref.at[a_rows(i)],
                send_sem=sy_sems.at[i % 2],
                recv_sem=ry_sems.at[i],
                device_id=peer_y,
                device_id_type=pl.DeviceIdType.MESH,
            )

        def reduce_and_forward(j):
            x_rdma(j).wait_recv()
            ld = pltpu.make_async_copy(
                recv_x.at[pl.ds(j * BM, BM)], th_v, th_sem
            )
            ld.start()
            if j >= 2:
                y_rdma(j - 2).wait_send()
                pltpu.make_async_copy(
                    ys_v.at[(j - 2) % 2], o_ref.at[a_rows(j - 2)],
                    st_sems.at[(j - 2) % 2],
                ).wait()
            ld.wait()
            ys_v[j % 2] = pt_v[j % 2] + th_v[...]
            y_rdma(j).start()
            pltpu.make_async_copy(
                ys_v.at[j % 2], o_ref.at[a_rows(j)], st_sems.at[j % 2]
            ).start()

        ld0 = pltpu.make_async_copy(a_ref.at[a_rows(0)], a_v.at[0], a_sems.at[0])
        ld0.start()

        for i in range(nt):
            if i >= 2:
                x_rdma(i - 2).wait_send()
            pltpu.make_async_copy(
                a_ref.at[a_rows(i)], a_v.at[i % 2], a_sems.at[i % 2]
            ).wait()
            if i + 1 < nt:
                pltpu.make_async_copy(
                    a_ref.at[a_rows(i + 1)], a_v.at[(i + 1) % 2],
                    a_sems.at[(i + 1) % 2],
                ).start()
            pt_v[i % 2] = jnp.dot(
                a_v[i % 2], b_ref[...], preferred_element_type=jnp.float32
            )
            x_rdma(i).start()
            if i >= 1:
                reduce_and_forward(i - 1)
        reduce_and_forward(nt - 1)

        x_rdma(nt - 2).wait_send()
        x_rdma(nt - 1).wait_send()
        y_rdma(nt - 2).wait_send()
        y_rdma(nt - 1).wait_send()
        for s in (0, 1):
            pltpu.make_async_copy(
                ys_v.at[s], o_ref.at[a_rows(nt - 2 + s)], st_sems.at[s]
            ).wait()
        for i in range(nt):
            y_rdma(i).wait_recv()

    out, _recv = pl.pallas_call(
        body,
        in_specs=[
            pl.BlockSpec(memory_space=pltpu.MemorySpace.HBM),
            pl.BlockSpec(memory_space=pltpu.MemorySpace.VMEM),
        ],
        out_specs=[
            pl.BlockSpec(memory_space=pltpu.MemorySpace.HBM),
            pl.BlockSpec(memory_space=pltpu.MemorySpace.HBM),
        ],
        out_shape=[
            jax.ShapeDtypeStruct((M, N), jnp.float32),
            jax.ShapeDtypeStruct((half, N), jnp.float32),
        ],
        scratch_shapes=[
            pltpu.MemorySpace.VMEM((2, BM, K), jnp.float32),
            pltpu.MemorySpace.VMEM((2, BM, N), jnp.float32),
            pltpu.MemorySpace.VMEM((BM, N), jnp.float32),
            pltpu.MemorySpace.VMEM((2, BM, N), jnp.float32),
            pltpu.SemaphoreType.DMA((2,)),
            pltpu.SemaphoreType.DMA,
            pltpu.SemaphoreType.DMA((2,)),
            pltpu.SemaphoreType.DMA((2,)),
            pltpu.SemaphoreType.DMA((nt,)),
            pltpu.SemaphoreType.DMA((2,)),
            pltpu.SemaphoreType.DMA((nt,)),
        ],
        compiler_params=pltpu.CompilerParams(
            collective_id=0,
            vmem_limit_bytes=56 * 1024 * 1024,
        ),
    )(A, B)
    return out
